# Initial kernel scaffold; baseline (speedup 1.0000x reference)
import jax
import jax.numpy as jnp
from jax.experimental import pallas as pl


def kernel(x, edge_index, W1, b1, W2, b2):
    raise NotImplementedError("write your pallas kernel here")


# speedup vs baseline: 16.4257x; 16.4257x over previous
import functools

import jax
import jax.numpy as jnp
from jax import lax
from jax.experimental import pallas as pl
from jax.experimental.pallas import tpu as pltpu
from jax.experimental.pallas import tpu_sc as plsc

N_NODES = 10000
NPAD = 10240
N_EDGES = 320000
CHUNK = 128
NROWS = 2560
EPAD = NROWS * CHUNK
NWORKERS = 32
ROWS_PER_TILE = NROWS // NWORKERS
HID = 64

_mesh = plsc.VectorSubcoreMesh(core_axis_name="c", subcore_axis_name="s")
_sc_params = pltpu.CompilerParams(use_tc_tiling_on_sc=False)


@functools.partial(
    pl.kernel,
    out_type=jax.ShapeDtypeStruct((2, NPAD), jnp.float32),
    mesh=_mesh,
    compiler_params=_sc_params,
    scratch_types=[
        pltpu.VMEM((ROWS_PER_TILE, CHUNK), jnp.int32),
        pltpu.VMEM((NPAD // 16,), jnp.float32),
        pltpu.VMEM((CHUNK,), jnp.float32),
        pltpu.VMEM_SHARED((NPAD,), jnp.float32),
    ],
)
def _deg_kernel(dst_hbm, deg_out, dstbuf, zbuf, ones, dacc):
    cid = lax.axis_index("c")
    sid = lax.axis_index("s")
    wid = sid * 2 + cid

    def zfill(i, _):
        zbuf[pl.ds(i * 16, 16)] = jnp.zeros((16,), jnp.float32)
        return 0

    lax.fori_loop(0, (NPAD // 16) // 16, zfill, 0)

    def ofill(i, _):
        ones[pl.ds(i * 16, 16)] = jnp.ones((16,), jnp.float32)
        return 0

    lax.fori_loop(0, CHUNK // 16, ofill, 0)

    seg = NPAD // 16
    pltpu.sync_copy(zbuf, dacc.at[pl.ds(sid * seg, seg)])
    pltpu.sync_copy(dst_hbm.at[pl.ds(wid * ROWS_PER_TILE, ROWS_PER_TILE)], dstbuf)
    plsc.subcore_barrier()

    def chunk(c, _):
        pltpu.sync_copy(ones, dacc.at[dstbuf.at[c]], add=True)
        return 0

    lax.fori_loop(0, ROWS_PER_TILE, chunk, 0)
    plsc.subcore_barrier()
    pltpu.sync_copy(dacc.at[pl.ds(sid * seg, seg)],
                    deg_out.at[cid, pl.ds(sid * seg, seg)])


@functools.partial(
    pl.kernel,
    out_type=jax.ShapeDtypeStruct((2, NPAD, HID), jnp.float32),
    mesh=_mesh,
    compiler_params=_sc_params,
    scratch_types=[
        pltpu.VMEM((ROWS_PER_TILE, CHUNK), jnp.int32),
        pltpu.VMEM((ROWS_PER_TILE, CHUNK), jnp.int32),
        pltpu.VMEM((CHUNK, HID), jnp.float32),
        pltpu.VMEM((CHUNK, HID), jnp.float32),
        pltpu.VMEM_SHARED((NPAD, HID), jnp.float32),
        pltpu.SemaphoreType.DMA,
        pltpu.SemaphoreType.DMA,
    ],
)
def _scatter_kernel(g_hbm, src_hbm, dst_hbm, out_hbm,
                    srcbuf, dstbuf, rows0, rows1, acc, sem0, sem1):
    cid = lax.axis_index("c")
    sid = lax.axis_index("s")
    wid = sid * 2 + cid
    rows = (rows0, rows1)
    sems = (sem0, sem1)

    def zrow(r, _):
        def zcol(k, _):
            rows0[r, pl.ds(k * 16, 16)] = jnp.zeros((16,), jnp.float32)
            return 0
        return lax.fori_loop(0, HID // 16, zcol, 0)

    lax.fori_loop(0, CHUNK, zrow, 0)
    seg = NPAD // 16
    for k in range(seg // CHUNK):
        pltpu.sync_copy(rows0, acc.at[pl.ds(sid * seg + k * CHUNK, CHUNK)])

    pltpu.sync_copy(src_hbm.at[pl.ds(wid * ROWS_PER_TILE, ROWS_PER_TILE)], srcbuf)
    pltpu.sync_copy(dst_hbm.at[pl.ds(wid * ROWS_PER_TILE, ROWS_PER_TILE)], dstbuf)
    plsc.subcore_barrier()

    def fire(c, b):
        pltpu.async_copy(g_hbm.at[srcbuf.at[c]], rows[b], sems[b])

    def wait(c, b):
        pltpu.make_async_copy(g_hbm.at[srcbuf.at[c]], rows[b], sems[b]).wait()

    def scat(c, b):
        pltpu.sync_copy(rows[b], acc.at[dstbuf.at[c]], add=True)

    fire(0, 0)

    def body(i, _):
        c0 = 2 * i
        fire(c0 + 1, 1)
        wait(c0, 0)
        scat(c0, 0)

        @pl.when(c0 + 2 < ROWS_PER_TILE)
        def _():
            fire(c0 + 2, 0)

        wait(c0 + 1, 1)
        scat(c0 + 1, 1)
        return 0

    lax.fori_loop(0, ROWS_PER_TILE // 2, body, 0)
    plsc.subcore_barrier()
    pltpu.sync_copy(acc.at[pl.ds(sid * seg, seg)],
                    out_hbm.at[cid, pl.ds(sid * seg, seg)])


RB = 1000


def _tc1_body(deg_ref, x_ref, w_ref, o_ref):
    d = deg_ref[...]
    dinv = lax.rsqrt(d[0] + d[1] + 1.0)
    o_ref[...] = jnp.dot(x_ref[...], w_ref[...],
                         preferred_element_type=jnp.float32) * dinv


def _tc2_body(deg_ref, s_ref, g_ref, b_ref, o_ref):
    d = deg_ref[...]
    dinv = lax.rsqrt(d[0] + d[1] + 1.0)
    s = s_ref[...]
    z = jnp.maximum(dinv * (s[0] + s[1] + g_ref[...]) + b_ref[...], 0.0)
    o_ref[...] = dinv * z


def _tc3_body(deg_ref, s_ref, g_ref, w_ref, b_ref, o_ref):
    d = deg_ref[...]
    dinv = lax.rsqrt(d[0] + d[1] + 1.0)
    s = s_ref[...]
    t = dinv * (s[0] + s[1] + g_ref[...])
    o_ref[...] = jnp.dot(t, w_ref[...],
                         preferred_element_type=jnp.float32) + b_ref[...]


_deg_spec = pl.BlockSpec((2, RB, 1), lambda i: (0, i, 0))
_s_spec = pl.BlockSpec((2, RB, HID), lambda i: (0, i, 0))

_tc1 = pl.pallas_call(
    _tc1_body,
    grid=(N_NODES // RB,),
    in_specs=[_deg_spec,
              pl.BlockSpec((RB, 128), lambda i: (i, 0)),
              pl.BlockSpec((128, HID), lambda i: (0, 0))],
    out_specs=pl.BlockSpec((RB, HID), lambda i: (i, 0)),
    out_shape=jax.ShapeDtypeStruct((N_NODES, HID), jnp.float32),
)

_tc2 = pl.pallas_call(
    _tc2_body,
    grid=(N_NODES // RB,),
    in_specs=[_deg_spec,
              _s_spec,
              pl.BlockSpec((RB, HID), lambda i: (i, 0)),
              pl.BlockSpec((1, HID), lambda i: (0, 0))],
    out_specs=pl.BlockSpec((RB, HID), lambda i: (i, 0)),
    out_shape=jax.ShapeDtypeStruct((N_NODES, HID), jnp.float32),
)

_tc3 = pl.pallas_call(
    _tc3_body,
    grid=(N_NODES // RB,),
    in_specs=[_deg_spec,
              _s_spec,
              pl.BlockSpec((RB, HID), lambda i: (i, 0)),
              pl.BlockSpec((HID, 128), lambda i: (0, 0)),
              pl.BlockSpec((1, 128), lambda i: (0, 0))],
    out_specs=pl.BlockSpec((RB, 128), lambda i: (i, 0)),
    out_shape=jax.ShapeDtypeStruct((N_NODES, 128), jnp.float32),
)


def kernel(x, edge_index, W1, b1, W2, b2):
    src = edge_index[0].astype(jnp.int32)
    dst = edge_index[1].astype(jnp.int32)
    pad = EPAD - N_EDGES
    src_p = jnp.concatenate([src, jnp.zeros((pad,), jnp.int32)]).reshape(NROWS, CHUNK)
    dst_p = jnp.concatenate([dst, jnp.full((pad,), N_NODES, jnp.int32)]).reshape(NROWS, CHUNK)

    deg2 = _deg_kernel(dst_p)
    deg3 = deg2.reshape(2, NPAD, 1)

    g1 = _tc1(deg3, x, W1)
    s1 = _scatter_kernel(g1, src_p, dst_p)
    g2 = _tc2(deg3, s1, g1, b1.reshape(1, HID))
    s2 = _scatter_kernel(g2, src_p, dst_p)
    return _tc3(deg3, s2, g2, W2, b2.reshape(1, 128))
